# Initial kernel scaffold; baseline (speedup 1.0000x reference)
import jax
import jax.numpy as jnp
from jax.experimental import pallas as pl


def kernel(position_ids, embed_table):
    raise NotImplementedError("write your pallas kernel here")


# speedup vs baseline: 1.8234x; 1.8234x over previous
import functools

import jax
import jax.numpy as jnp
from jax import lax
from jax.experimental import pallas as pl
from jax.experimental.pallas import tpu as pltpu
from jax.experimental.pallas import tpu_sc as plsc

_DIM = 2048
_NC, _NS = 2, 16
_NW = _NC * _NS
_C = 16


def kernel(position_ids, embed_table):
    b, s = position_ids.shape
    n = b * s
    b_per_w = n // _NW
    nch = b_per_w // _C
    idx = position_ids.reshape(n).astype(jnp.int32)
    mesh = plsc.VectorSubcoreMesh(core_axis_name="c", subcore_axis_name="s")

    @functools.partial(
        pl.kernel,
        out_type=jax.ShapeDtypeStruct((n, _DIM), jnp.float32),
        mesh=mesh,
        scratch_types=[
            pltpu.VMEM((b_per_w,), jnp.int32),
            pltpu.VMEM((_C, _DIM), jnp.float32),
            pltpu.VMEM((_C, _DIM), jnp.float32),
            pltpu.SemaphoreType.DMA,
            pltpu.SemaphoreType.DMA,
            pltpu.SemaphoreType.DMA,
            pltpu.SemaphoreType.DMA,
        ],
    )
    def gather_rows(table_hbm, idx_hbm, out_hbm,
                    idx_v, buf0, buf1, gs0, gs1, ws0, ws1):
        wid = lax.axis_index("s") * _NC + lax.axis_index("c")
        base = wid * b_per_w
        pltpu.sync_copy(idx_hbm.at[pl.ds(base, b_per_w)], idx_v)

        bufs = (buf0, buf1)
        gsems = (gs0, gs1)
        wsems = (ws0, ws1)

        def start_gather(g, slot):
            pltpu.async_copy(
                table_hbm.at[idx_v.at[pl.ds(g * _C, _C)]], bufs[slot], gsems[slot])

        def wait_gather(slot):
            pltpu.make_async_copy(
                table_hbm.at[idx_v.at[pl.ds(0, _C)]], bufs[slot], gsems[slot]).wait()

        def start_write(g, slot):
            pltpu.async_copy(
                bufs[slot], out_hbm.at[pl.ds(base + g * _C, _C)], wsems[slot])

        def wait_write(slot):
            pltpu.make_async_copy(
                bufs[slot], out_hbm.at[pl.ds(base, _C)], wsems[slot]).wait()

        start_gather(0, 0)

        @pl.loop(0, nch, step=2)
        def _(g0):
            for k in range(2):
                g = g0 + k
                slot, other = k, 1 - k

                @pl.when(g + 1 < nch)
                def _():
                    @pl.when(g >= 1)
                    def _():
                        wait_write(other)

                    start_gather(g + 1, other)

                wait_gather(slot)
                start_write(g, slot)

        wait_write((nch - 1) % 2)

    return gather_rows(embed_table, idx).reshape(b, s, _DIM)


# speedup vs baseline: 1.8463x; 1.0125x over previous
import functools

import jax
import jax.numpy as jnp
from jax import lax
from jax.experimental import pallas as pl
from jax.experimental.pallas import tpu as pltpu
from jax.experimental.pallas import tpu_sc as plsc

_DIM = 2048
_NC, _NS = 2, 16
_NW = _NC * _NS
_C = 8
_D = 4
_P = 3


def kernel(position_ids, embed_table):
    b, s = position_ids.shape
    n = b * s
    b_per_w = n // _NW
    nch = b_per_w // _C
    idx = position_ids.reshape(n).astype(jnp.int32)
    mesh = plsc.VectorSubcoreMesh(core_axis_name="c", subcore_axis_name="s")

    @functools.partial(
        pl.kernel,
        out_type=jax.ShapeDtypeStruct((n, _DIM), jnp.float32),
        mesh=mesh,
        scratch_types=(
            [pltpu.VMEM((b_per_w,), jnp.int32)]
            + [pltpu.VMEM((_C, _DIM), jnp.float32)] * _D
            + [pltpu.SemaphoreType.DMA] * (2 * _D)
        ),
    )
    def gather_rows(table_hbm, idx_hbm, out_hbm, idx_v, *scratch):
        bufs = scratch[:_D]
        gsems = scratch[_D:2 * _D]
        wsems = scratch[2 * _D:]
        wid = lax.axis_index("s") * _NC + lax.axis_index("c")
        base = wid * b_per_w
        pltpu.sync_copy(idx_hbm.at[pl.ds(base, b_per_w)], idx_v)

        def start_gather(g, slot):
            pltpu.async_copy(
                table_hbm.at[idx_v.at[pl.ds(g * _C, _C)]], bufs[slot], gsems[slot])

        def wait_gather(slot):
            pltpu.make_async_copy(
                table_hbm.at[idx_v.at[pl.ds(0, _C)]], bufs[slot], gsems[slot]).wait()

        def start_write(g, slot):
            pltpu.async_copy(
                bufs[slot], out_hbm.at[pl.ds(base + g * _C, _C)], wsems[slot])

        def wait_write(slot):
            pltpu.make_async_copy(
                bufs[slot], out_hbm.at[pl.ds(base, _C)], wsems[slot]).wait()

        for j in range(_P):
            start_gather(j, j)

        @pl.loop(0, nch, step=_D)
        def _(g0):
            for k in range(_D):
                g = g0 + k
                slot = k
                pf_slot = (k + _P) % _D

                @pl.when(g + _P < nch)
                def _():
                    @pl.when(g + _P - _D >= 0)
                    def _():
                        wait_write(pf_slot)

                    start_gather(g + _P, pf_slot)

                wait_gather(slot)
                start_write(g, slot)

        for i in range(_D):
            wait_write((nch - _D + i) % _D)

    return gather_rows(embed_table, idx).reshape(b, s, _DIM)
